# Initial kernel scaffold; baseline (speedup 1.0000x reference)
import jax
import jax.numpy as jnp
from jax.experimental import pallas as pl


def kernel(x, edge_index, W1l, W1r, a1, b1, Wlin, blin, W2l, W2r, a2, b2):
    raise NotImplementedError("write your pallas kernel here")


# speedup vs baseline: 3.0712x; 3.0712x over previous
import functools

import jax
import jax.numpy as jnp
from jax import lax
from jax.experimental import pallas as pl
from jax.experimental.pallas import tpu as pltpu
from jax.experimental.pallas import tpu_sc as plsc

N = 10000
E = 160000
HEADS = 2
D = 512
NSLAB = 4
SLAB = 128
NPAD = 10240
EPAD = 172032
CH_A = EPAD // 32
NSTRIPE = NPAD // 16
NEG = -1.0e30


def _mm(x, W, b, act_in):
    M, K = x.shape
    Nc = W.shape[1]
    BM = 512
    b2 = b.reshape(1, Nc)
    relu_in = act_in is not None
    ain = act_in.reshape(1, K) if relu_in else jnp.zeros((1, K), jnp.float32)

    def body(x_ref, w_ref, b_ref, a_ref, o_ref):
        xv = x_ref[...]
        if relu_in:
            xv = jnp.maximum(xv + a_ref[...], 0.0)
        acc = jnp.dot(xv, w_ref[...], preferred_element_type=jnp.float32)
        o_ref[...] = acc + b_ref[...]

    return pl.pallas_call(
        body,
        grid=(pl.cdiv(M, BM),),
        in_specs=[
            pl.BlockSpec((BM, K), lambda i: (i, 0)),
            pl.BlockSpec((K, Nc), lambda i: (0, 0)),
            pl.BlockSpec((1, Nc), lambda i: (0, 0)),
            pl.BlockSpec((1, K), lambda i: (0, 0)),
        ],
        out_specs=pl.BlockSpec((BM, Nc), lambda i: (i, 0)),
        out_shape=jax.ShapeDtypeStruct((M, Nc), jnp.float32),
    )(x, W, b2, ain)


def _sc_logits(xl_slabs, xr_slabs, s, d, att):
    mesh = plsc.VectorSubcoreMesh(core_axis_name="c", subcore_axis_name="s")

    @functools.partial(
        pl.kernel,
        mesh=mesh,
        compiler_params=pltpu.CompilerParams(needs_layout_passes=False),
        out_type=[
            jax.ShapeDtypeStruct((EPAD,), jnp.float32),
            jax.ShapeDtypeStruct((EPAD,), jnp.float32),
            jax.ShapeDtypeStruct((2, 2, NPAD), jnp.float32),
        ],
        scratch_types=[
            pltpu.VMEM((CH_A,), jnp.int32),
            pltpu.VMEM((CH_A,), jnp.int32),
            pltpu.VMEM((CH_A,), jnp.float32),
            pltpu.VMEM((CH_A,), jnp.float32),
            pltpu.VMEM((NPAD,), jnp.float32),
            pltpu.VMEM((NPAD,), jnp.float32),
            [pltpu.VMEM((64, SLAB), jnp.float32) for _ in range(4)],
            [pltpu.VMEM((64, SLAB), jnp.float32) for _ in range(4)],
            pltpu.VMEM((D,), jnp.float32),
            pltpu.VMEM((NSTRIPE,), jnp.float32),
            pltpu.VMEM((NSTRIPE,), jnp.float32),
            pltpu.VMEM((64,), jnp.int32),
            pltpu.VMEM((64,), jnp.int32),
            pltpu.VMEM_SHARED((16, NPAD), jnp.float32),
            pltpu.SemaphoreType.DMA,
            pltpu.SemaphoreType.DMA,
        ],
    )
    def k(xl0, xl1, xl2, xl3, xr0, xr1, xr2, xr3, s_hbm, d_hbm, att_hbm,
          lg0_hbm, lg1_hbm, mpart_hbm,
          s_v, d_v, lg0_v, lg1_v, mloc0, mloc1, bufl, bufr, att_v,
          red_acc, red_tmp, sbuf64, dbuf64, stage, sem0, sem1):
        cid = lax.axis_index("c")
        sid = lax.axis_index("s")
        wid = sid * 2 + cid
        base = wid * CH_A
        xls = (xl0, xl1, xl2, xl3)
        xrs = (xr0, xr1, xr2, xr3)
        mlocs = (mloc0, mloc1)
        lgvs = (lg0_v, lg1_v)
        lghs = (lg0_hbm, lg1_hbm)

        pltpu.sync_copy(s_hbm.at[pl.ds(base, CH_A)], s_v)
        pltpu.sync_copy(d_hbm.at[pl.ds(base, CH_A)], d_v)
        pltpu.sync_copy(att_hbm, att_v)

        iota = jnp.arange(16, dtype=jnp.int32)
        evecs = [iota + 16 * g for g in range(4)]
        ones16 = jnp.ones((16,), jnp.int32)

        def minit(i, _):
            mloc0[pl.ds(i * 16, 16)] = jnp.full((16,), NEG, jnp.float32)
            mloc1[pl.ds(i * 16, 16)] = jnp.full((16,), NEG, jnp.float32)
            return 0
        lax.fori_loop(0, NPAD // 16, minit, 0)

        def chunk(ci, _):
            off = ci * 64
            for g in range(4):
                sbuf64[pl.ds(16 * g, 16)] = s_v[pl.ds(off + 16 * g, 16)]
                dbuf64[pl.ds(16 * g, 16)] = d_v[pl.ds(off + 16 * g, 16)]
            cps = []
            for kk in range(4):
                cps.append(pltpu.async_copy(xls[kk].at[sbuf64], bufl[kk], sem0))
                cps.append(pltpu.async_copy(xrs[kk].at[dbuf64], bufr[kk], sem1))
            for cp in cps:
                cp.wait()
            for h in range(HEADS):
                accs = (jnp.zeros((16,), jnp.float32),) * 4
                for kk in (2 * h, 2 * h + 1):
                    def cbody(c, accs, kk=kk):
                        csp = jnp.full((16,), c, jnp.int32)
                        attb = plsc.load_gather(att_v, [csp + (kk * SLAB)])
                        out = []
                        for g in range(4):
                            vl = plsc.load_gather(bufl[kk], [evecs[g], csp])
                            vr = plsc.load_gather(bufr[kk], [evecs[g], csp])
                            t = vl + vr
                            out.append(accs[g] + jnp.maximum(t, 0.2 * t) * attb)
                        return tuple(out)
                    accs = lax.fori_loop(0, SLAB, cbody, accs)
                mloc = mlocs[h]
                for g in range(4):
                    lgvs[h][pl.ds(off + 16 * g, 16)] = accs[g]
                    dvec = d_v[pl.ds(off + 16 * g, 16)]
                    val = accs[g]

                    def wbody(pend, dvec=dvec, val=val, mloc=mloc):
                        msk = pend != 0
                        cur = plsc.load_gather(mloc, [dvec])
                        new = jnp.maximum(cur, val)
                        plsc.store_scatter(mloc, [dvec], new, mask=msk)
                        chk = plsc.load_gather(mloc, [dvec])
                        return jnp.where(msk & (chk < new), 1, 0).astype(jnp.int32)

                    lax.while_loop(lambda p: jnp.max(p, axis=0) > 0, wbody, ones16)
            return 0
        lax.fori_loop(0, CH_A // 64, chunk, 0)

        pltpu.sync_copy(lg0_v, lg0_hbm.at[pl.ds(base, CH_A)])
        pltpu.sync_copy(lg1_v, lg1_hbm.at[pl.ds(base, CH_A)])

        for h in range(HEADS):
            plsc.subcore_barrier()
            pltpu.sync_copy(mlocs[h], stage.at[sid])
            plsc.subcore_barrier()
            rbase = sid * NSTRIPE
            pltpu.sync_copy(stage.at[0, pl.ds(rbase, NSTRIPE)], red_acc)

            def tred(t, _):
                pltpu.sync_copy(stage.at[t, pl.ds(rbase, NSTRIPE)], red_tmp)

                def rstep(j, _):
                    red_acc[pl.ds(j * 16, 16)] = jnp.maximum(
                        red_acc[pl.ds(j * 16, 16)], red_tmp[pl.ds(j * 16, 16)])
                    return 0
                lax.fori_loop(0, NSTRIPE // 16, rstep, 0)
                return 0
            lax.fori_loop(1, 16, tred, 0)
            pltpu.sync_copy(red_acc, mpart_hbm.at[cid, h, pl.ds(rbase, NSTRIPE)])

    return k(*xl_slabs, *xr_slabs, s, d, att)


CH_D = EPAD // 16


def _sc_alpha(d, lg0, lg1, m_part4):
    mesh = plsc.VectorSubcoreMesh(core_axis_name="c", subcore_axis_name="s")
    NR = NPAD // 128

    @functools.partial(
        pl.kernel,
        mesh=mesh,
        compiler_params=pltpu.CompilerParams(needs_layout_passes=False),
        out_type=jax.ShapeDtypeStruct((2, EPAD), jnp.float32),
        scratch_types=[
            pltpu.VMEM((CH_D,), jnp.float32),
            pltpu.VMEM((NR, 128), jnp.float32),
            pltpu.VMEM((NR, 128), jnp.float32),
            pltpu.VMEM((64,), jnp.int32),
            pltpu.VMEM((64,), jnp.float32),
            pltpu.VMEM((NR,), jnp.int32),
            pltpu.VMEM_SHARED((NR, 128), jnp.float32),
        ],
    )
    def k(d_hbm, lg0_hbm, lg1_hbm, mpart_hbm, alpha_hbm,
          a_v, m_v, zloc, dbuf, lgbuf, idrows, zfin_sh):
        cid = lax.axis_index("c")
        sid = lax.axis_index("s")
        base = sid * CH_D
        lghs = (lg0_hbm, lg1_hbm)
        iota = jnp.arange(16, dtype=jnp.int32)

        def idf(i, _):
            idrows[pl.ds(i * 16, 16)] = iota + i * 16
            return 0
        lax.fori_loop(0, NR // 16, idf, 0)

        for h in range(HEADS):
            @pl.when(cid == h)
            def _(h=h):
                pltpu.sync_copy(mpart_hbm.at[0, h], m_v)
                pltpu.sync_copy(mpart_hbm.at[1, h], zloc)

                def mmax(j, _):
                    def mc(i, _):
                        m_v[j, pl.ds(i * 16, 16)] = jnp.maximum(
                            m_v[j, pl.ds(i * 16, 16)], zloc[j, pl.ds(i * 16, 16)])
                        return 0
                    lax.fori_loop(0, 8, mc, 0)
                    return 0
                lax.fori_loop(0, NR, mmax, 0)

                def zz(j, _):
                    def zc(i, _):
                        zloc[j, pl.ds(i * 16, 16)] = jnp.zeros((16,), jnp.float32)
                        return 0
                    lax.fori_loop(0, 8, zc, 0)
                    return 0
                lax.fori_loop(0, NR, zz, 0)

                @pl.when(sid == 0)
                def _():
                    pltpu.sync_copy(zloc, zfin_sh)
                plsc.subcore_barrier()

                def abody(ci, _):
                    off = ci * 64
                    pltpu.sync_copy(d_hbm.at[pl.ds(base + off, 64)], dbuf)
                    pltpu.sync_copy(lghs[h].at[pl.ds(base + off, 64)], lgbuf)
                    for g in range(4):
                        dvec = dbuf[pl.ds(16 * g, 16)]
                        hi = lax.shift_right_logical(dvec, 7)
                        lo = lax.bitwise_and(dvec, 127)
                        lg = lgbuf[pl.ds(16 * g, 16)]
                        mg = plsc.load_gather(m_v, [hi, lo])
                        a = jnp.exp(lg - mg)
                        a_v[pl.ds(off + 16 * g, 16)] = a
                        plsc.addupdate_scatter(zloc, [hi, lo], a)
                    return 0
                lax.fori_loop(0, CH_D // 64, abody, 0)

                pltpu.sync_copy(zloc, zfin_sh.at[idrows], add=True)
                plsc.subcore_barrier()
                pltpu.sync_copy(zfin_sh, m_v)

                def albody(ci, _):
                    off = ci * 64
                    pltpu.sync_copy(d_hbm.at[pl.ds(base + off, 64)], dbuf)
                    for g in range(4):
                        dvec = dbuf[pl.ds(16 * g, 16)]
                        hi = lax.shift_right_logical(dvec, 7)
                        lo = lax.bitwise_and(dvec, 127)
                        zg = plsc.load_gather(m_v, [hi, lo])
                        o = off + 16 * g
                        a_v[pl.ds(o, 16)] = a_v[pl.ds(o, 16)] / (zg + 1e-16)
                    return 0
                lax.fori_loop(0, CH_D // 64, albody, 0)
                pltpu.sync_copy(a_v, alpha_hbm.at[h, pl.ds(base, CH_D)])

    return k(d, lg0, lg1, m_part4)


SUP = 1536
NSUP = CH_D // SUP
CPS = SUP // 64


def _sc_scatter(xl_slabs, s, d, alpha):
    mesh = plsc.VectorSubcoreMesh(core_axis_name="c", subcore_axis_name="s")

    @functools.partial(
        pl.kernel,
        mesh=mesh,
        compiler_params=pltpu.CompilerParams(needs_layout_passes=False),
        out_type=[jax.ShapeDtypeStruct((NPAD, SLAB), jnp.float32)
                  for _ in range(NSLAB)],
        scratch_types=[
            [pltpu.VMEM((64, SLAB), jnp.float32) for _ in range(2)],
            [pltpu.VMEM((64,), jnp.int32) for _ in range(2)],
            [pltpu.VMEM((64,), jnp.int32) for _ in range(2)],
            pltpu.VMEM((SUP,), jnp.int32),
            pltpu.VMEM((SUP,), jnp.int32),
            pltpu.VMEM((SUP,), jnp.float32),
            pltpu.VMEM_SHARED((NPAD, SLAB), jnp.float32),
            pltpu.SemaphoreType.DMA,
            pltpu.SemaphoreType.DMA,
        ],
    )
    def k(xl0, xl1, xl2, xl3, s_hbm, d_hbm, alpha_hbm,
          o0, o1, o2, o3,
          rowbuf, sb64, db64, sbig, dbig, abig, out_sh, gsem, ssem):
        cid = lax.axis_index("c")
        sid = lax.axis_index("s")
        base = sid * CH_D
        rbase = sid * NSTRIPE
        xls = (xl0, xl1, xl2, xl3)
        outs = (o0, o1, o2, o3)
        iota = jnp.arange(16, dtype=jnp.int32)
        evecs = [iota + 16 * g for g in range(4)]

        for h in range(HEADS):
            @pl.when(cid == h)
            def _(h=h):
                for kk in (2 * h, 2 * h + 1):
                    def zrow(r, _):
                        def zc(j, _):
                            rowbuf[0][r, pl.ds(j * 16, 16)] = jnp.zeros(
                                (16,), jnp.float32)
                            return 0
                        lax.fori_loop(0, SLAB // 16, zc, 0)
                        return 0
                    lax.fori_loop(0, 64, zrow, 0)

                    def zcp(r, _):
                        pltpu.sync_copy(rowbuf[0],
                                        out_sh.at[pl.ds(rbase + r * 64, 64)])
                        return 0
                    lax.fori_loop(0, NSTRIPE // 64, zcp, 0)
                    plsc.subcore_barrier()

                    def fire(i, p, kk=kk):
                        off = i * 64
                        for g in range(4):
                            sb64[p][pl.ds(16 * g, 16)] = sbig[pl.ds(off + 16 * g, 16)]
                            db64[p][pl.ds(16 * g, 16)] = dbig[pl.ds(off + 16 * g, 16)]
                        return pltpu.async_copy(xls[kk].at[sb64[p]], rowbuf[p],
                                                gsem)

                    def proc(i, p):
                        off = i * 64
                        als = [abig[pl.ds(off + 16 * g, 16)] for g in range(4)]

                        def cbody(c, _):
                            csp = jnp.full((16,), c, jnp.int32)
                            for g in range(4):
                                v = plsc.load_gather(rowbuf[p], [evecs[g], csp])
                                plsc.store_scatter(rowbuf[p], [evecs[g], csp],
                                                   v * als[g])
                            return 0
                        lax.fori_loop(0, SLAB, cbody, 0)
                        return pltpu.async_copy(rowbuf[p], out_sh.at[db64[p]],
                                                ssem, add=True)

                    def drain_s(p, kk=kk):
                        pltpu.make_async_copy(rowbuf[p], out_sh.at[db64[p]],
                                              ssem).wait()

                    def wait_g(p, kk=kk):
                        pltpu.make_async_copy(xls[kk].at[sb64[p]], rowbuf[p],
                                              gsem).wait()

                    def step(i, p, first=False, last=False):
                        if not first:
                            drain_s(1 - p)
                        if not last:
                            fire(i + 1, 1 - p)
                        wait_g(p)
                        proc(i, p)

                    def super_body(sup, _, kk=kk):
                        soff = sup * SUP
                        pltpu.sync_copy(s_hbm.at[pl.ds(base + soff, SUP)], sbig)
                        pltpu.sync_copy(d_hbm.at[pl.ds(base + soff, SUP)], dbig)
                        pltpu.sync_copy(alpha_hbm.at[h, pl.ds(base + soff, SUP)],
                                        abig)
                        fire(0, 0)
                        step(0, 0, first=True)
                        step(1, 1)

                        def pair(cj, _):
                            step(cj * 2, 0)
                            step(cj * 2 + 1, 1)
                            return 0
                        lax.fori_loop(1, CPS // 2 - 1, pair, 0)
                        step(CPS - 2, 0)
                        step(CPS - 1, 1, last=True)
                        drain_s(1)
                        return 0
                    lax.fori_loop(0, NSUP, super_body, 0)
                    plsc.subcore_barrier()
                    pltpu.sync_copy(out_sh.at[pl.ds(rbase, NSTRIPE)],
                                    outs[kk].at[pl.ds(rbase, NSTRIPE)])
                    plsc.subcore_barrier()

    return k(*xl_slabs, s, d, alpha)


def _bias_add(x, b):
    M, Nc = x.shape

    def body(x_ref, b_ref, o_ref):
        o_ref[...] = x_ref[...] + b_ref[...]

    return pl.pallas_call(
        body,
        grid=(pl.cdiv(M, 1024),),
        in_specs=[
            pl.BlockSpec((1024, Nc), lambda i: (i, 0)),
            pl.BlockSpec((1, Nc), lambda i: (0, 0)),
        ],
        out_specs=pl.BlockSpec((1024, Nc), lambda i: (i, 0)),
        out_shape=jax.ShapeDtypeStruct((M, Nc), jnp.float32),
    )(x, b.reshape(1, Nc))


def _pad_rows(a, rows):
    return jnp.pad(a, ((0, rows - a.shape[0]), (0, 0)))


def _slabs(a):
    ap = _pad_rows(a, NPAD)
    return [ap[:, i * SLAB:(i + 1) * SLAB] for i in range(NSLAB)]


def _slabs64(a):
    ap = _pad_rows(a, NPAD)
    hs = SLAB // 2
    return [ap[:, i * hs:(i + 1) * hs] for i in range(2 * NSLAB)]


def _gat_sc(xl, xr, s, d, att):
    xl_s = _slabs(xl)
    xr_s = _slabs(xr)
    attf = att.reshape(D)
    lg0, lg1, m_part = _sc_logits(xl_s, xr_s, s, d, attf)
    alpha = _sc_alpha(d, lg0, lg1, m_part.reshape(2, 2, NPAD // 128, 128))
    outs = _sc_scatter(xl_s, s, d, alpha)
    return jnp.concatenate(outs, axis=1)[:N]


def kernel(x, edge_index, W1l, W1r, a1, b1, Wlin, blin, W2l, W2r, a2, b2):
    src, dst = edge_index[0], edge_index[1]
    loop = jnp.arange(N, dtype=src.dtype)
    npad_e = EPAD - (E + N)
    s = jnp.concatenate([src, loop, jnp.zeros((npad_e,), src.dtype)])
    d = jnp.concatenate([dst, loop, jnp.full((npad_e,), N, dst.dtype)])
    s = s.astype(jnp.int32)
    d = d.astype(jnp.int32)

    zeros512 = jnp.zeros((D,), jnp.float32)
    xl1 = _mm(x, W1l, zeros512, None)
    xr1 = _mm(x, W1r, zeros512, None)
    g1 = _gat_sc(xl1, xr1, s, d, a1)
    h2 = _mm(g1, Wlin, blin, b1)
    xl2 = _mm(h2, W2l, zeros512, None)
    xr2 = _mm(h2, W2r, zeros512, None)
    g2 = _gat_sc(xl2, xr2, s, d, a2)
    return _bias_add(g2, b2)


# speedup vs baseline: 3.0756x; 1.0014x over previous
import functools

import jax
import jax.numpy as jnp
from jax import lax
from jax.experimental import pallas as pl
from jax.experimental.pallas import tpu as pltpu
from jax.experimental.pallas import tpu_sc as plsc

N = 10000
E = 160000
HEADS = 2
D = 512
NSLAB = 4
SLAB = 128
NPAD = 10240
EPAD = 172032
CH_A = EPAD // 32
NSTRIPE = NPAD // 16
NEG = -1.0e30


def _mm(x, W, b, act_in):
    M, K = x.shape
    Nc = W.shape[1]
    BM = 512
    b2 = b.reshape(1, Nc)
    relu_in = act_in is not None
    ain = act_in.reshape(1, K) if relu_in else jnp.zeros((1, K), jnp.float32)

    def body(x_ref, w_ref, b_ref, a_ref, o_ref):
        xv = x_ref[...]
        if relu_in:
            xv = jnp.maximum(xv + a_ref[...], 0.0)
        acc = jnp.dot(xv, w_ref[...], preferred_element_type=jnp.float32)
        o_ref[...] = acc + b_ref[...]

    return pl.pallas_call(
        body,
        grid=(pl.cdiv(M, BM),),
        in_specs=[
            pl.BlockSpec((BM, K), lambda i: (i, 0)),
            pl.BlockSpec((K, Nc), lambda i: (0, 0)),
            pl.BlockSpec((1, Nc), lambda i: (0, 0)),
            pl.BlockSpec((1, K), lambda i: (0, 0)),
        ],
        out_specs=pl.BlockSpec((BM, Nc), lambda i: (i, 0)),
        out_shape=jax.ShapeDtypeStruct((M, Nc), jnp.float32),
    )(x, W, b2, ain)


def _sc_logits(xl_slabs, xr_slabs, s, d, att):
    mesh = plsc.VectorSubcoreMesh(core_axis_name="c", subcore_axis_name="s")

    @functools.partial(
        pl.kernel,
        mesh=mesh,
        compiler_params=pltpu.CompilerParams(needs_layout_passes=False),
        out_type=[
            jax.ShapeDtypeStruct((EPAD,), jnp.float32),
            jax.ShapeDtypeStruct((EPAD,), jnp.float32),
            jax.ShapeDtypeStruct((2, 2, NPAD), jnp.float32),
        ],
        scratch_types=[
            pltpu.VMEM((CH_A,), jnp.int32),
            pltpu.VMEM((CH_A,), jnp.int32),
            pltpu.VMEM((CH_A,), jnp.float32),
            pltpu.VMEM((CH_A,), jnp.float32),
            pltpu.VMEM((NPAD,), jnp.float32),
            pltpu.VMEM((NPAD,), jnp.float32),
            [pltpu.VMEM((64, SLAB), jnp.float32) for _ in range(4)],
            [pltpu.VMEM((64, SLAB), jnp.float32) for _ in range(4)],
            pltpu.VMEM((D,), jnp.float32),
            pltpu.VMEM((NSTRIPE,), jnp.float32),
            pltpu.VMEM((NSTRIPE,), jnp.float32),
            pltpu.VMEM((64,), jnp.int32),
            pltpu.VMEM((64,), jnp.int32),
            pltpu.VMEM_SHARED((16, NPAD), jnp.float32),
            pltpu.SemaphoreType.DMA,
            pltpu.SemaphoreType.DMA,
        ],
    )
    def k(xl0, xl1, xl2, xl3, xr0, xr1, xr2, xr3, s_hbm, d_hbm, att_hbm,
          lg0_hbm, lg1_hbm, mpart_hbm,
          s_v, d_v, lg0_v, lg1_v, mloc0, mloc1, bufl, bufr, att_v,
          red_acc, red_tmp, sbuf64, dbuf64, stage, sem0, sem1):
        cid = lax.axis_index("c")
        sid = lax.axis_index("s")
        wid = sid * 2 + cid
        base = wid * CH_A
        xls = (xl0, xl1, xl2, xl3)
        xrs = (xr0, xr1, xr2, xr3)
        mlocs = (mloc0, mloc1)
        lgvs = (lg0_v, lg1_v)
        lghs = (lg0_hbm, lg1_hbm)

        pltpu.sync_copy(s_hbm.at[pl.ds(base, CH_A)], s_v)
        pltpu.sync_copy(d_hbm.at[pl.ds(base, CH_A)], d_v)
        pltpu.sync_copy(att_hbm, att_v)

        iota = jnp.arange(16, dtype=jnp.int32)
        evecs = [iota + 16 * g for g in range(4)]
        ones16 = jnp.ones((16,), jnp.int32)

        def minit(i, _):
            mloc0[pl.ds(i * 16, 16)] = jnp.full((16,), NEG, jnp.float32)
            mloc1[pl.ds(i * 16, 16)] = jnp.full((16,), NEG, jnp.float32)
            return 0
        lax.fori_loop(0, NPAD // 16, minit, 0)

        def chunk(ci, _):
            off = ci * 64
            for g in range(4):
                sbuf64[pl.ds(16 * g, 16)] = s_v[pl.ds(off + 16 * g, 16)]
                dbuf64[pl.ds(16 * g, 16)] = d_v[pl.ds(off + 16 * g, 16)]
            cps = []
            for kk in range(4):
                cps.append(pltpu.async_copy(xls[kk].at[sbuf64], bufl[kk], sem0))
                cps.append(pltpu.async_copy(xrs[kk].at[dbuf64], bufr[kk], sem1))
            for cp in cps:
                cp.wait()
            for h in range(HEADS):
                accs = (jnp.zeros((16,), jnp.float32),) * 4
                for kk in (2 * h, 2 * h + 1):
                    def cbody(c, accs, kk=kk):
                        csp = jnp.full((16,), c, jnp.int32)
                        attb = plsc.load_gather(att_v, [csp + (kk * SLAB)])
                        out = []
                        for g in range(4):
                            vl = plsc.load_gather(bufl[kk], [evecs[g], csp])
                            vr = plsc.load_gather(bufr[kk], [evecs[g], csp])
                            t = vl + vr
                            out.append(accs[g] + jnp.maximum(t, 0.2 * t) * attb)
                        return tuple(out)
                    accs = lax.fori_loop(0, SLAB, cbody, accs)
                mloc = mlocs[h]
                for g in range(4):
                    lgvs[h][pl.ds(off + 16 * g, 16)] = accs[g]
                    dvec = d_v[pl.ds(off + 16 * g, 16)]
                    val = accs[g]

                    def wbody(pend, dvec=dvec, val=val, mloc=mloc):
                        msk = pend != 0
                        cur = plsc.load_gather(mloc, [dvec])
                        new = jnp.maximum(cur, val)
                        plsc.store_scatter(mloc, [dvec], new, mask=msk)
                        chk = plsc.load_gather(mloc, [dvec])
                        return jnp.where(msk & (chk < new), 1, 0).astype(jnp.int32)

                    lax.while_loop(lambda p: jnp.max(p, axis=0) > 0, wbody, ones16)
            return 0
        lax.fori_loop(0, CH_A // 64, chunk, 0)

        pltpu.sync_copy(lg0_v, lg0_hbm.at[pl.ds(base, CH_A)])
        pltpu.sync_copy(lg1_v, lg1_hbm.at[pl.ds(base, CH_A)])

        for h in range(HEADS):
            plsc.subcore_barrier()
            pltpu.sync_copy(mlocs[h], stage.at[sid])
            plsc.subcore_barrier()
            rbase = sid * NSTRIPE
            pltpu.sync_copy(stage.at[0, pl.ds(rbase, NSTRIPE)], red_acc)

            def tred(t, _):
                pltpu.sync_copy(stage.at[t, pl.ds(rbase, NSTRIPE)], red_tmp)

                def rstep(j, _):
                    red_acc[pl.ds(j * 16, 16)] = jnp.maximum(
                        red_acc[pl.ds(j * 16, 16)], red_tmp[pl.ds(j * 16, 16)])
                    return 0
                lax.fori_loop(0, NSTRIPE // 16, rstep, 0)
                return 0
            lax.fori_loop(1, 16, tred, 0)
            pltpu.sync_copy(red_acc, mpart_hbm.at[cid, h, pl.ds(rbase, NSTRIPE)])

    return k(*xl_slabs, *xr_slabs, s, d, att)


CH_D = EPAD // 16


def _sc_alpha(d, lg0, lg1, m_part4):
    mesh = plsc.VectorSubcoreMesh(core_axis_name="c", subcore_axis_name="s")
    NR = NPAD // 128

    @functools.partial(
        pl.kernel,
        mesh=mesh,
        compiler_params=pltpu.CompilerParams(needs_layout_passes=False),
        out_type=jax.ShapeDtypeStruct((2, EPAD), jnp.float32),
        scratch_types=[
            pltpu.VMEM((CH_D,), jnp.float32),
            pltpu.VMEM((NR, 128), jnp.float32),
            pltpu.VMEM((NR, 128), jnp.float32),
            pltpu.VMEM((64,), jnp.int32),
            pltpu.VMEM((64,), jnp.float32),
            pltpu.VMEM((NR,), jnp.int32),
            pltpu.VMEM_SHARED((NR, 128), jnp.float32),
        ],
    )
    def k(d_hbm, lg0_hbm, lg1_hbm, mpart_hbm, alpha_hbm,
          a_v, m_v, zloc, dbuf, lgbuf, idrows, zfin_sh):
        cid = lax.axis_index("c")
        sid = lax.axis_index("s")
        base = sid * CH_D
        lghs = (lg0_hbm, lg1_hbm)
        iota = jnp.arange(16, dtype=jnp.int32)

        def idf(i, _):
            idrows[pl.ds(i * 16, 16)] = iota + i * 16
            return 0
        lax.fori_loop(0, NR // 16, idf, 0)

        for h in range(HEADS):
            @pl.when(cid == h)
            def _(h=h):
                pltpu.sync_copy(mpart_hbm.at[0, h], m_v)
                pltpu.sync_copy(mpart_hbm.at[1, h], zloc)

                def mmax(j, _):
                    def mc(i, _):
                        m_v[j, pl.ds(i * 16, 16)] = jnp.maximum(
                            m_v[j, pl.ds(i * 16, 16)], zloc[j, pl.ds(i * 16, 16)])
                        return 0
                    lax.fori_loop(0, 8, mc, 0)
                    return 0
                lax.fori_loop(0, NR, mmax, 0)

                def zz(j, _):
                    def zc(i, _):
                        zloc[j, pl.ds(i * 16, 16)] = jnp.zeros((16,), jnp.float32)
                        return 0
                    lax.fori_loop(0, 8, zc, 0)
                    return 0
                lax.fori_loop(0, NR, zz, 0)

                @pl.when(sid == 0)
                def _():
                    pltpu.sync_copy(zloc, zfin_sh)
                plsc.subcore_barrier()

                def abody(ci, _):
                    off = ci * 64
                    pltpu.sync_copy(d_hbm.at[pl.ds(base + off, 64)], dbuf)
                    pltpu.sync_copy(lghs[h].at[pl.ds(base + off, 64)], lgbuf)
                    for g in range(4):
                        dvec = dbuf[pl.ds(16 * g, 16)]
                        hi = lax.shift_right_logical(dvec, 7)
                        lo = lax.bitwise_and(dvec, 127)
                        lg = lgbuf[pl.ds(16 * g, 16)]
                        mg = plsc.load_gather(m_v, [hi, lo])
                        a = jnp.exp(lg - mg)
                        a_v[pl.ds(off + 16 * g, 16)] = a
                        plsc.addupdate_scatter(zloc, [hi, lo], a)
                    return 0
                lax.fori_loop(0, CH_D // 64, abody, 0)

                pltpu.sync_copy(zloc, zfin_sh.at[idrows], add=True)
                plsc.subcore_barrier()
                pltpu.sync_copy(zfin_sh, m_v)

                def albody(ci, _):
                    off = ci * 64
                    pltpu.sync_copy(d_hbm.at[pl.ds(base + off, 64)], dbuf)
                    for g in range(4):
                        dvec = dbuf[pl.ds(16 * g, 16)]
                        hi = lax.shift_right_logical(dvec, 7)
                        lo = lax.bitwise_and(dvec, 127)
                        zg = plsc.load_gather(m_v, [hi, lo])
                        o = off + 16 * g
                        a_v[pl.ds(o, 16)] = a_v[pl.ds(o, 16)] / (zg + 1e-16)
                    return 0
                lax.fori_loop(0, CH_D // 64, albody, 0)
                pltpu.sync_copy(a_v, alpha_hbm.at[h, pl.ds(base, CH_D)])

    return k(d, lg0, lg1, m_part4)


SUP = 1536
NSUP = CH_D // SUP
CHK = 128
CPS = SUP // CHK


def _sc_scatter(xl_slabs, s, d, alpha):
    mesh = plsc.VectorSubcoreMesh(core_axis_name="c", subcore_axis_name="s")

    @functools.partial(
        pl.kernel,
        mesh=mesh,
        compiler_params=pltpu.CompilerParams(needs_layout_passes=False),
        out_type=[jax.ShapeDtypeStruct((NPAD, SLAB), jnp.float32)
                  for _ in range(NSLAB)],
        scratch_types=[
            [pltpu.VMEM((CHK, SLAB), jnp.float32) for _ in range(2)],
            [pltpu.VMEM((CHK,), jnp.int32) for _ in range(2)],
            [pltpu.VMEM((CHK,), jnp.int32) for _ in range(2)],
            pltpu.VMEM((SUP,), jnp.int32),
            pltpu.VMEM((SUP,), jnp.int32),
            pltpu.VMEM((SUP,), jnp.float32),
            pltpu.VMEM_SHARED((NPAD, SLAB), jnp.float32),
            pltpu.SemaphoreType.DMA,
            pltpu.SemaphoreType.DMA,
        ],
    )
    def k(xl0, xl1, xl2, xl3, s_hbm, d_hbm, alpha_hbm,
          o0, o1, o2, o3,
          rowbuf, sb64, db64, sbig, dbig, abig, out_sh, gsem, ssem):
        cid = lax.axis_index("c")
        sid = lax.axis_index("s")
        base = sid * CH_D
        rbase = sid * NSTRIPE
        xls = (xl0, xl1, xl2, xl3)
        outs = (o0, o1, o2, o3)
        iota = jnp.arange(16, dtype=jnp.int32)
        evecs = [iota + 16 * g for g in range(CHK // 16)]

        for h in range(HEADS):
            @pl.when(cid == h)
            def _(h=h):
                for kk in (2 * h, 2 * h + 1):
                    def zrow(r, _):
                        def zc(j, _):
                            rowbuf[0][r, pl.ds(j * 16, 16)] = jnp.zeros(
                                (16,), jnp.float32)
                            return 0
                        lax.fori_loop(0, SLAB // 16, zc, 0)
                        return 0
                    lax.fori_loop(0, CHK, zrow, 0)

                    def zcp(r, _):
                        pltpu.sync_copy(rowbuf[0],
                                        out_sh.at[pl.ds(rbase + r * CHK, CHK)])
                        return 0
                    lax.fori_loop(0, NSTRIPE // CHK, zcp, 0)
                    plsc.subcore_barrier()

                    def fire(i, p, kk=kk):
                        off = i * CHK
                        for g in range(CHK // 16):
                            sb64[p][pl.ds(16 * g, 16)] = sbig[pl.ds(off + 16 * g, 16)]
                            db64[p][pl.ds(16 * g, 16)] = dbig[pl.ds(off + 16 * g, 16)]
                        return pltpu.async_copy(xls[kk].at[sb64[p]], rowbuf[p],
                                                gsem)

                    def proc(i, p):
                        off = i * CHK
                        als = [abig[pl.ds(off + 16 * g, 16)]
                               for g in range(CHK // 16)]

                        def cbody(c, _):
                            csp = jnp.full((16,), c, jnp.int32)
                            for g in range(CHK // 16):
                                v = plsc.load_gather(rowbuf[p], [evecs[g], csp])
                                plsc.store_scatter(rowbuf[p], [evecs[g], csp],
                                                   v * als[g])
                            return 0
                        lax.fori_loop(0, SLAB, cbody, 0)
                        return pltpu.async_copy(rowbuf[p], out_sh.at[db64[p]],
                                                ssem, add=True)

                    def drain_s(p, kk=kk):
                        pltpu.make_async_copy(rowbuf[p], out_sh.at[db64[p]],
                                              ssem).wait()

                    def wait_g(p, kk=kk):
                        pltpu.make_async_copy(xls[kk].at[sb64[p]], rowbuf[p],
                                              gsem).wait()

                    def step(i, p, first=False, last=False):
                        if not first:
                            drain_s(1 - p)
                        if not last:
                            fire(i + 1, 1 - p)
                        wait_g(p)
                        proc(i, p)

                    def super_body(sup, _, kk=kk):
                        soff = sup * SUP
                        pltpu.sync_copy(s_hbm.at[pl.ds(base + soff, SUP)], sbig)
                        pltpu.sync_copy(d_hbm.at[pl.ds(base + soff, SUP)], dbig)
                        pltpu.sync_copy(alpha_hbm.at[h, pl.ds(base + soff, SUP)],
                                        abig)
                        fire(0, 0)
                        step(0, 0, first=True)
                        step(1, 1)

                        def pair(cj, _):
                            step(cj * 2, 0)
                            step(cj * 2 + 1, 1)
                            return 0
                        lax.fori_loop(1, CPS // 2 - 1, pair, 0)
                        step(CPS - 2, 0)
                        step(CPS - 1, 1, last=True)
                        drain_s(1)
                        return 0
                    lax.fori_loop(0, NSUP, super_body, 0)
                    plsc.subcore_barrier()
                    pltpu.sync_copy(out_sh.at[pl.ds(rbase, NSTRIPE)],
                                    outs[kk].at[pl.ds(rbase, NSTRIPE)])
                    plsc.subcore_barrier()

    return k(*xl_slabs, s, d, alpha)


def _bias_add(x, b):
    M, Nc = x.shape

    def body(x_ref, b_ref, o_ref):
        o_ref[...] = x_ref[...] + b_ref[...]

    return pl.pallas_call(
        body,
        grid=(pl.cdiv(M, 1024),),
        in_specs=[
            pl.BlockSpec((1024, Nc), lambda i: (i, 0)),
            pl.BlockSpec((1, Nc), lambda i: (0, 0)),
        ],
        out_specs=pl.BlockSpec((1024, Nc), lambda i: (i, 0)),
        out_shape=jax.ShapeDtypeStruct((M, Nc), jnp.float32),
    )(x, b.reshape(1, Nc))


def _pad_rows(a, rows):
    return jnp.pad(a, ((0, rows - a.shape[0]), (0, 0)))


def _slabs(a):
    ap = _pad_rows(a, NPAD)
    return [ap[:, i * SLAB:(i + 1) * SLAB] for i in range(NSLAB)]


def _slabs64(a):
    ap = _pad_rows(a, NPAD)
    hs = SLAB // 2
    return [ap[:, i * hs:(i + 1) * hs] for i in range(2 * NSLAB)]


def _gat_sc(xl, xr, s, d, att):
    xl_s = _slabs(xl)
    xr_s = _slabs(xr)
    attf = att.reshape(D)
    lg0, lg1, m_part = _sc_logits(xl_s, xr_s, s, d, attf)
    alpha = _sc_alpha(d, lg0, lg1, m_part.reshape(2, 2, NPAD // 128, 128))
    outs = _sc_scatter(xl_s, s, d, alpha)
    return jnp.concatenate(outs, axis=1)[:N]


def kernel(x, edge_index, W1l, W1r, a1, b1, Wlin, blin, W2l, W2r, a2, b2):
    src, dst = edge_index[0], edge_index[1]
    loop = jnp.arange(N, dtype=src.dtype)
    npad_e = EPAD - (E + N)
    s = jnp.concatenate([src, loop, jnp.zeros((npad_e,), src.dtype)])
    d = jnp.concatenate([dst, loop, jnp.full((npad_e,), N, dst.dtype)])
    s = s.astype(jnp.int32)
    d = d.astype(jnp.int32)

    zeros512 = jnp.zeros((D,), jnp.float32)
    xl1 = _mm(x, W1l, zeros512, None)
    xr1 = _mm(x, W1r, zeros512, None)
    g1 = _gat_sc(xl1, xr1, s, d, a1)
    h2 = _mm(g1, Wlin, blin, b1)
    xl2 = _mm(h2, W2l, zeros512, None)
    xr2 = _mm(h2, W2r, zeros512, None)
    g2 = _gat_sc(xl2, xr2, s, d, a2)
    return _bias_add(g2, b2)


# speedup vs baseline: 3.1193x; 1.0142x over previous
import functools

import jax
import jax.numpy as jnp
from jax import lax
from jax.experimental import pallas as pl
from jax.experimental.pallas import tpu as pltpu
from jax.experimental.pallas import tpu_sc as plsc

N = 10000
E = 160000
HEADS = 2
D = 512
NSLAB = 4
SLAB = 128
NPAD = 10240
EPAD = 172032
CH_A = EPAD // 32
NSTRIPE = NPAD // 16
NEG = -1.0e30


def _mm(x, W, b, act_in):
    M, K = x.shape
    Nc = W.shape[1]
    BM = 512
    b2 = b.reshape(1, Nc)
    relu_in = act_in is not None
    ain = act_in.reshape(1, K) if relu_in else jnp.zeros((1, K), jnp.float32)

    def body(x_ref, w_ref, b_ref, a_ref, o_ref):
        xv = x_ref[...]
        if relu_in:
            xv = jnp.maximum(xv + a_ref[...], 0.0)
        acc = jnp.dot(xv, w_ref[...], preferred_element_type=jnp.float32)
        o_ref[...] = acc + b_ref[...]

    return pl.pallas_call(
        body,
        grid=(pl.cdiv(M, BM),),
        in_specs=[
            pl.BlockSpec((BM, K), lambda i: (i, 0)),
            pl.BlockSpec((K, Nc), lambda i: (0, 0)),
            pl.BlockSpec((1, Nc), lambda i: (0, 0)),
            pl.BlockSpec((1, K), lambda i: (0, 0)),
        ],
        out_specs=pl.BlockSpec((BM, Nc), lambda i: (i, 0)),
        out_shape=jax.ShapeDtypeStruct((M, Nc), jnp.float32),
    )(x, W, b2, ain)


def _sc_logits(xl_slabs, xr_slabs, s, d, att):
    mesh = plsc.VectorSubcoreMesh(core_axis_name="c", subcore_axis_name="s")

    @functools.partial(
        pl.kernel,
        mesh=mesh,
        compiler_params=pltpu.CompilerParams(needs_layout_passes=False),
        out_type=[
            jax.ShapeDtypeStruct((EPAD,), jnp.float32),
            jax.ShapeDtypeStruct((EPAD,), jnp.float32),
            jax.ShapeDtypeStruct((2, 2, NPAD), jnp.float32),
        ],
        scratch_types=[
            pltpu.VMEM((CH_A,), jnp.int32),
            pltpu.VMEM((CH_A,), jnp.int32),
            pltpu.VMEM((CH_A,), jnp.float32),
            pltpu.VMEM((CH_A,), jnp.float32),
            pltpu.VMEM((NPAD,), jnp.float32),
            pltpu.VMEM((NPAD,), jnp.float32),
            [pltpu.VMEM((64, SLAB), jnp.float32) for _ in range(4)],
            [pltpu.VMEM((64, SLAB), jnp.float32) for _ in range(4)],
            pltpu.VMEM((D,), jnp.float32),
            pltpu.VMEM((NSTRIPE,), jnp.float32),
            pltpu.VMEM((NSTRIPE,), jnp.float32),
            pltpu.VMEM((64,), jnp.int32),
            pltpu.VMEM((64,), jnp.int32),
            pltpu.VMEM_SHARED((16, NPAD), jnp.float32),
            pltpu.SemaphoreType.DMA,
            pltpu.SemaphoreType.DMA,
        ],
    )
    def k(xl0, xl1, xl2, xl3, xr0, xr1, xr2, xr3, s_hbm, d_hbm, att_hbm,
          lg0_hbm, lg1_hbm, mpart_hbm,
          s_v, d_v, lg0_v, lg1_v, mloc0, mloc1, bufl, bufr, att_v,
          red_acc, red_tmp, sbuf64, dbuf64, stage, sem0, sem1):
        cid = lax.axis_index("c")
        sid = lax.axis_index("s")
        wid = sid * 2 + cid
        base = wid * CH_A
        xls = (xl0, xl1, xl2, xl3)
        xrs = (xr0, xr1, xr2, xr3)
        mlocs = (mloc0, mloc1)
        lgvs = (lg0_v, lg1_v)
        lghs = (lg0_hbm, lg1_hbm)

        pltpu.sync_copy(s_hbm.at[pl.ds(base, CH_A)], s_v)
        pltpu.sync_copy(d_hbm.at[pl.ds(base, CH_A)], d_v)
        pltpu.sync_copy(att_hbm, att_v)

        iota = jnp.arange(16, dtype=jnp.int32)
        evecs = [iota + 16 * g for g in range(4)]
        ones16 = jnp.ones((16,), jnp.int32)

        def minit(i, _):
            mloc0[pl.ds(i * 16, 16)] = jnp.full((16,), NEG, jnp.float32)
            mloc1[pl.ds(i * 16, 16)] = jnp.full((16,), NEG, jnp.float32)
            return 0
        lax.fori_loop(0, NPAD // 16, minit, 0)

        def chunk(ci, _):
            off = ci * 64
            for g in range(4):
                sbuf64[pl.ds(16 * g, 16)] = s_v[pl.ds(off + 16 * g, 16)]
                dbuf64[pl.ds(16 * g, 16)] = d_v[pl.ds(off + 16 * g, 16)]
            cps = []
            for kk in range(4):
                cps.append(pltpu.async_copy(xls[kk].at[sbuf64], bufl[kk], sem0))
                cps.append(pltpu.async_copy(xrs[kk].at[dbuf64], bufr[kk], sem1))
            for cp in cps:
                cp.wait()
            for h in range(HEADS):
                accs = (jnp.zeros((16,), jnp.float32),) * 4
                for kk in (2 * h, 2 * h + 1):
                    def cbody(c, accs, kk=kk):
                        csp = jnp.full((16,), c, jnp.int32)
                        attb = plsc.load_gather(att_v, [csp + (kk * SLAB)])
                        out = []
                        for g in range(4):
                            vl = plsc.load_gather(bufl[kk], [evecs[g], csp])
                            vr = plsc.load_gather(bufr[kk], [evecs[g], csp])
                            t = vl + vr
                            out.append(accs[g] + jnp.maximum(t, 0.2 * t) * attb)
                        return tuple(out)
                    accs = lax.fori_loop(0, SLAB, cbody, accs)
                mloc = mlocs[h]
                for g in range(4):
                    lgvs[h][pl.ds(off + 16 * g, 16)] = accs[g]
                    dvec = d_v[pl.ds(off + 16 * g, 16)]
                    val = accs[g]

                    def wbody(pend, dvec=dvec, val=val, mloc=mloc):
                        msk = pend != 0
                        cur = plsc.load_gather(mloc, [dvec])
                        new = jnp.maximum(cur, val)
                        plsc.store_scatter(mloc, [dvec], new, mask=msk)
                        chk = plsc.load_gather(mloc, [dvec])
                        return jnp.where(msk & (chk < new), 1, 0).astype(jnp.int32)

                    lax.while_loop(lambda p: jnp.max(p, axis=0) > 0, wbody, ones16)
            return 0
        lax.fori_loop(0, CH_A // 64, chunk, 0)

        pltpu.sync_copy(lg0_v, lg0_hbm.at[pl.ds(base, CH_A)])
        pltpu.sync_copy(lg1_v, lg1_hbm.at[pl.ds(base, CH_A)])

        for h in range(HEADS):
            plsc.subcore_barrier()
            pltpu.sync_copy(mlocs[h], stage.at[sid])
            plsc.subcore_barrier()
            rbase = sid * NSTRIPE
            pltpu.sync_copy(stage.at[0, pl.ds(rbase, NSTRIPE)], red_acc)

            def tred(t, _):
                pltpu.sync_copy(stage.at[t, pl.ds(rbase, NSTRIPE)], red_tmp)

                def rstep(j, _):
                    red_acc[pl.ds(j * 16, 16)] = jnp.maximum(
                        red_acc[pl.ds(j * 16, 16)], red_tmp[pl.ds(j * 16, 16)])
                    return 0
                lax.fori_loop(0, NSTRIPE // 16, rstep, 0)
                return 0
            lax.fori_loop(1, 16, tred, 0)
            pltpu.sync_copy(red_acc, mpart_hbm.at[cid, h, pl.ds(rbase, NSTRIPE)])

    return k(*xl_slabs, *xr_slabs, s, d, att)


CH_D = EPAD // 16


def _sc_alpha(d, lg0, lg1, m_part4):
    mesh = plsc.VectorSubcoreMesh(core_axis_name="c", subcore_axis_name="s")
    NR = NPAD // 128

    @functools.partial(
        pl.kernel,
        mesh=mesh,
        compiler_params=pltpu.CompilerParams(needs_layout_passes=False),
        out_type=jax.ShapeDtypeStruct((2, EPAD), jnp.float32),
        scratch_types=[
            pltpu.VMEM((CH_D,), jnp.float32),
            pltpu.VMEM((NR, 128), jnp.float32),
            pltpu.VMEM((NR, 128), jnp.float32),
            pltpu.VMEM((64,), jnp.int32),
            pltpu.VMEM((64,), jnp.float32),
            pltpu.VMEM((NR,), jnp.int32),
            pltpu.VMEM_SHARED((NR, 128), jnp.float32),
        ],
    )
    def k(d_hbm, lg0_hbm, lg1_hbm, mpart_hbm, alpha_hbm,
          a_v, m_v, zloc, dbuf, lgbuf, idrows, zfin_sh):
        cid = lax.axis_index("c")
        sid = lax.axis_index("s")
        base = sid * CH_D
        lghs = (lg0_hbm, lg1_hbm)
        iota = jnp.arange(16, dtype=jnp.int32)

        def idf(i, _):
            idrows[pl.ds(i * 16, 16)] = iota + i * 16
            return 0
        lax.fori_loop(0, NR // 16, idf, 0)

        for h in range(HEADS):
            @pl.when(cid == h)
            def _(h=h):
                pltpu.sync_copy(mpart_hbm.at[0, h], m_v)
                pltpu.sync_copy(mpart_hbm.at[1, h], zloc)

                def mmax(j, _):
                    def mc(i, _):
                        m_v[j, pl.ds(i * 16, 16)] = jnp.maximum(
                            m_v[j, pl.ds(i * 16, 16)], zloc[j, pl.ds(i * 16, 16)])
                        return 0
                    lax.fori_loop(0, 8, mc, 0)
                    return 0
                lax.fori_loop(0, NR, mmax, 0)

                def zz(j, _):
                    def zc(i, _):
                        zloc[j, pl.ds(i * 16, 16)] = jnp.zeros((16,), jnp.float32)
                        return 0
                    lax.fori_loop(0, 8, zc, 0)
                    return 0
                lax.fori_loop(0, NR, zz, 0)

                @pl.when(sid == 0)
                def _():
                    pltpu.sync_copy(zloc, zfin_sh)
                plsc.subcore_barrier()

                def abody(ci, _):
                    off = ci * 64
                    pltpu.sync_copy(d_hbm.at[pl.ds(base + off, 64)], dbuf)
                    pltpu.sync_copy(lghs[h].at[pl.ds(base + off, 64)], lgbuf)
                    for g in range(4):
                        dvec = dbuf[pl.ds(16 * g, 16)]
                        hi = lax.shift_right_logical(dvec, 7)
                        lo = lax.bitwise_and(dvec, 127)
                        lg = lgbuf[pl.ds(16 * g, 16)]
                        mg = plsc.load_gather(m_v, [hi, lo])
                        a = jnp.exp(lg - mg)
                        a_v[pl.ds(off + 16 * g, 16)] = a
                        plsc.addupdate_scatter(zloc, [hi, lo], a)
                    return 0
                lax.fori_loop(0, CH_D // 64, abody, 0)

                pltpu.sync_copy(zloc, zfin_sh.at[idrows], add=True)
                plsc.subcore_barrier()
                pltpu.sync_copy(zfin_sh, m_v)

                def albody(ci, _):
                    off = ci * 64
                    pltpu.sync_copy(d_hbm.at[pl.ds(base + off, 64)], dbuf)
                    for g in range(4):
                        dvec = dbuf[pl.ds(16 * g, 16)]
                        hi = lax.shift_right_logical(dvec, 7)
                        lo = lax.bitwise_and(dvec, 127)
                        zg = plsc.load_gather(m_v, [hi, lo])
                        o = off + 16 * g
                        a_v[pl.ds(o, 16)] = a_v[pl.ds(o, 16)] / (zg + 1e-16)
                    return 0
                lax.fori_loop(0, CH_D // 64, albody, 0)
                pltpu.sync_copy(a_v, alpha_hbm.at[h, pl.ds(base, CH_D)])

    return k(d, lg0, lg1, m_part4)


SUP = 1536
NSUP = CH_D // SUP
CHK = 64
CPS = SUP // CHK


def _sc_scatter(xl_slabs, s, d, alpha):
    mesh = plsc.VectorSubcoreMesh(core_axis_name="c", subcore_axis_name="s")
    NG = CHK // 16

    @functools.partial(
        pl.kernel,
        mesh=mesh,
        compiler_params=pltpu.CompilerParams(needs_layout_passes=False),
        out_type=[jax.ShapeDtypeStruct((NPAD, SLAB), jnp.float32)
                  for _ in range(NSLAB)],
        scratch_types=[
            [pltpu.VMEM((CHK, SLAB), jnp.float32) for _ in range(2)],
            [pltpu.VMEM((CHK, SLAB), jnp.float32) for _ in range(2)],
            [pltpu.VMEM((CHK,), jnp.int32) for _ in range(2)],
            [pltpu.VMEM((CHK,), jnp.int32) for _ in range(2)],
            pltpu.VMEM((SUP,), jnp.int32),
            pltpu.VMEM((SUP,), jnp.int32),
            pltpu.VMEM((SUP,), jnp.float32),
            pltpu.VMEM_SHARED((NPAD, SLAB), jnp.float32),
            pltpu.SemaphoreType.DMA,
            pltpu.SemaphoreType.DMA,
        ],
    )
    def k(xl0, xl1, xl2, xl3, s_hbm, d_hbm, alpha_hbm,
          o0, o1, o2, o3,
          rowbuf, scbuf, sb64, db64, sbig, dbig, abig, out_sh, gsem, ssem):
        cid = lax.axis_index("c")
        sid = lax.axis_index("s")
        base = sid * CH_D
        rbase = sid * NSTRIPE
        xls = (xl0, xl1, xl2, xl3)
        outs = (o0, o1, o2, o3)
        iota = jnp.arange(16, dtype=jnp.int32)
        evecs = [iota + 16 * g for g in range(NG)]

        for h in range(HEADS):
            @pl.when(cid == h)
            def _(h=h):
                for kk in (2 * h, 2 * h + 1):
                    def zrow(r, _):
                        def zc(j, _):
                            rowbuf[0][r, pl.ds(j * 16, 16)] = jnp.zeros(
                                (16,), jnp.float32)
                            return 0
                        lax.fori_loop(0, SLAB // 16, zc, 0)
                        return 0
                    lax.fori_loop(0, CHK, zrow, 0)

                    def zcp(r, _):
                        pltpu.sync_copy(rowbuf[0],
                                        out_sh.at[pl.ds(rbase + r * CHK, CHK)])
                        return 0
                    lax.fori_loop(0, NSTRIPE // CHK, zcp, 0)
                    plsc.subcore_barrier()

                    def fire(i, p, kk=kk):
                        off = i * CHK
                        for g in range(NG):
                            sb64[p][pl.ds(16 * g, 16)] = sbig[pl.ds(off + 16 * g, 16)]
                        return pltpu.async_copy(xls[kk].at[sb64[p]], rowbuf[p],
                                                gsem)

                    def wait_g(p, kk=kk):
                        pltpu.make_async_copy(xls[kk].at[sb64[p]], rowbuf[p],
                                              gsem).wait()

                    def drain_s(p):
                        pltpu.make_async_copy(scbuf[p], out_sh.at[db64[p]],
                                              ssem).wait()

                    def step(i, p, first=False, last=False):
                        off = i * CHK
                        wait_g(p)
                        if not first:
                            drain_s(p)
                        als = [abig[pl.ds(off + 16 * g, 16)] for g in range(NG)]
                        for g in range(NG):
                            db64[p][pl.ds(16 * g, 16)] = dbig[pl.ds(off + 16 * g, 16)]

                        def cbody(c, _):
                            csp = jnp.full((16,), c, jnp.int32)
                            for g in range(NG):
                                v = plsc.load_gather(rowbuf[p], [evecs[g], csp])
                                plsc.store_scatter(scbuf[p], [evecs[g], csp],
                                                   v * als[g])
                            return 0
                        lax.fori_loop(0, SLAB, cbody, 0)
                        if not last:
                            fire(i + 2, p)
                        pltpu.async_copy(scbuf[p], out_sh.at[db64[p]], ssem,
                                         add=True)

                    def super_body(sup, _, kk=kk):
                        soff = sup * SUP
                        pltpu.sync_copy(s_hbm.at[pl.ds(base + soff, SUP)], sbig)
                        pltpu.sync_copy(d_hbm.at[pl.ds(base + soff, SUP)], dbig)
                        pltpu.sync_copy(alpha_hbm.at[h, pl.ds(base + soff, SUP)],
                                        abig)
                        fire(0, 0)
                        fire(1, 1)
                        step(0, 0, first=True)
                        step(1, 1, first=True)

                        def pair(cj, _):
                            step(cj * 2, 0)
                            step(cj * 2 + 1, 1)
                            return 0
                        lax.fori_loop(1, CPS // 2 - 1, pair, 0)
                        step(CPS - 2, 0, last=True)
                        step(CPS - 1, 1, last=True)
                        drain_s(0)
                        drain_s(1)
                        return 0
                    lax.fori_loop(0, NSUP, super_body, 0)
                    plsc.subcore_barrier()
                    pltpu.sync_copy(out_sh.at[pl.ds(rbase, NSTRIPE)],
                                    outs[kk].at[pl.ds(rbase, NSTRIPE)])
                    plsc.subcore_barrier()

    return k(*xl_slabs, s, d, alpha)


def _bias_add(x, b):
    M, Nc = x.shape

    def body(x_ref, b_ref, o_ref):
        o_ref[...] = x_ref[...] + b_ref[...]

    return pl.pallas_call(
        body,
        grid=(pl.cdiv(M, 1024),),
        in_specs=[
            pl.BlockSpec((1024, Nc), lambda i: (i, 0)),
            pl.BlockSpec((1, Nc), lambda i: (0, 0)),
        ],
        out_specs=pl.BlockSpec((1024, Nc), lambda i: (i, 0)),
        out_shape=jax.ShapeDtypeStruct((M, Nc), jnp.float32),
    )(x, b.reshape(1, Nc))


def _pad_rows(a, rows):
    return jnp.pad(a, ((0, rows - a.shape[0]), (0, 0)))


def _slabs(a):
    ap = _pad_rows(a, NPAD)
    return [ap[:, i * SLAB:(i + 1) * SLAB] for i in range(NSLAB)]


def _slabs64(a):
    ap = _pad_rows(a, NPAD)
    hs = SLAB // 2
    return [ap[:, i * hs:(i + 1) * hs] for i in range(2 * NSLAB)]


def _gat_sc(xl, xr, s, d, att):
    xl_s = _slabs(xl)
    xr_s = _slabs(xr)
    attf = att.reshape(D)
    lg0, lg1, m_part = _sc_logits(xl_s, xr_s, s, d, attf)
    alpha = _sc_alpha(d, lg0, lg1, m_part.reshape(2, 2, NPAD // 128, 128))
    outs = _sc_scatter(xl_s, s, d, alpha)
    return jnp.concatenate(outs, axis=1)[:N]


def kernel(x, edge_index, W1l, W1r, a1, b1, Wlin, blin, W2l, W2r, a2, b2):
    src, dst = edge_index[0], edge_index[1]
    loop = jnp.arange(N, dtype=src.dtype)
    npad_e = EPAD - (E + N)
    s = jnp.concatenate([src, loop, jnp.zeros((npad_e,), src.dtype)])
    d = jnp.concatenate([dst, loop, jnp.full((npad_e,), N, dst.dtype)])
    s = s.astype(jnp.int32)
    d = d.astype(jnp.int32)

    zeros512 = jnp.zeros((D,), jnp.float32)
    xl1 = _mm(x, W1l, zeros512, None)
    xr1 = _mm(x, W1r, zeros512, None)
    g1 = _gat_sc(xl1, xr1, s, d, a1)
    h2 = _mm(g1, Wlin, blin, b1)
    xl2 = _mm(h2, W2l, zeros512, None)
    xr2 = _mm(h2, W2r, zeros512, None)
    g2 = _gat_sc(xl2, xr2, s, d, a2)
    return _bias_add(g2, b2)


# speedup vs baseline: 3.1774x; 1.0186x over previous
import functools

import jax
import jax.numpy as jnp
from jax import lax
from jax.experimental import pallas as pl
from jax.experimental.pallas import tpu as pltpu
from jax.experimental.pallas import tpu_sc as plsc

N = 10000
E = 160000
HEADS = 2
D = 512
NSLAB = 4
SLAB = 128
NPAD = 10240
EPAD = 172032
CH_A = EPAD // 32
NSTRIPE = NPAD // 16
NEG = -1.0e30


def _mm(x, W, b, act_in):
    M, K = x.shape
    Nc = W.shape[1]
    BM = 512
    b2 = b.reshape(1, Nc)
    relu_in = act_in is not None
    ain = act_in.reshape(1, K) if relu_in else jnp.zeros((1, K), jnp.float32)

    def body(x_ref, w_ref, b_ref, a_ref, o_ref):
        xv = x_ref[...]
        if relu_in:
            xv = jnp.maximum(xv + a_ref[...], 0.0)
        acc = jnp.dot(xv, w_ref[...], preferred_element_type=jnp.float32)
        o_ref[...] = acc + b_ref[...]

    return pl.pallas_call(
        body,
        grid=(pl.cdiv(M, BM),),
        in_specs=[
            pl.BlockSpec((BM, K), lambda i: (i, 0)),
            pl.BlockSpec((K, Nc), lambda i: (0, 0)),
            pl.BlockSpec((1, Nc), lambda i: (0, 0)),
            pl.BlockSpec((1, K), lambda i: (0, 0)),
        ],
        out_specs=pl.BlockSpec((BM, Nc), lambda i: (i, 0)),
        out_shape=jax.ShapeDtypeStruct((M, Nc), jnp.float32),
    )(x, W, b2, ain)


def _sc_logits(xl_slabs, xr_slabs, s, d, att):
    mesh = plsc.VectorSubcoreMesh(core_axis_name="c", subcore_axis_name="s")

    @functools.partial(
        pl.kernel,
        mesh=mesh,
        compiler_params=pltpu.CompilerParams(needs_layout_passes=False),
        out_type=[
            jax.ShapeDtypeStruct((EPAD,), jnp.float32),
            jax.ShapeDtypeStruct((EPAD,), jnp.float32),
            jax.ShapeDtypeStruct((2, 2, NPAD), jnp.float32),
        ],
        scratch_types=[
            pltpu.VMEM((CH_A,), jnp.int32),
            pltpu.VMEM((CH_A,), jnp.int32),
            pltpu.VMEM((CH_A,), jnp.float32),
            pltpu.VMEM((CH_A,), jnp.float32),
            pltpu.VMEM((NPAD,), jnp.float32),
            pltpu.VMEM((NPAD,), jnp.float32),
            [pltpu.VMEM((64, SLAB), jnp.float32) for _ in range(4)],
            [pltpu.VMEM((64, SLAB), jnp.float32) for _ in range(4)],
            pltpu.VMEM((D,), jnp.float32),
            pltpu.VMEM((NSTRIPE,), jnp.float32),
            pltpu.VMEM((NSTRIPE,), jnp.float32),
            pltpu.VMEM((64,), jnp.int32),
            pltpu.VMEM((64,), jnp.int32),
            pltpu.VMEM_SHARED((16, NPAD), jnp.float32),
            pltpu.SemaphoreType.DMA,
            pltpu.SemaphoreType.DMA,
        ],
    )
    def k(xl0, xl1, xl2, xl3, xr0, xr1, xr2, xr3, s_hbm, d_hbm, att_hbm,
          lg0_hbm, lg1_hbm, mpart_hbm,
          s_v, d_v, lg0_v, lg1_v, mloc0, mloc1, bufl, bufr, att_v,
          red_acc, red_tmp, sbuf64, dbuf64, stage, sem0, sem1):
        cid = lax.axis_index("c")
        sid = lax.axis_index("s")
        wid = sid * 2 + cid
        base = wid * CH_A
        xls = (xl0, xl1, xl2, xl3)
        xrs = (xr0, xr1, xr2, xr3)
        mlocs = (mloc0, mloc1)
        lgvs = (lg0_v, lg1_v)
        lghs = (lg0_hbm, lg1_hbm)

        pltpu.sync_copy(s_hbm.at[pl.ds(base, CH_A)], s_v)
        pltpu.sync_copy(d_hbm.at[pl.ds(base, CH_A)], d_v)
        pltpu.sync_copy(att_hbm, att_v)

        iota = jnp.arange(16, dtype=jnp.int32)
        evecs = [iota + 16 * g for g in range(4)]
        ones16 = jnp.ones((16,), jnp.int32)

        def minit(i, _):
            mloc0[pl.ds(i * 16, 16)] = jnp.full((16,), NEG, jnp.float32)
            mloc1[pl.ds(i * 16, 16)] = jnp.full((16,), NEG, jnp.float32)
            return 0
        lax.fori_loop(0, NPAD // 16, minit, 0)

        def chunk(ci, _):
            off = ci * 64
            for g in range(4):
                sbuf64[pl.ds(16 * g, 16)] = s_v[pl.ds(off + 16 * g, 16)]
                dbuf64[pl.ds(16 * g, 16)] = d_v[pl.ds(off + 16 * g, 16)]
            cps = []
            for kk in range(4):
                cps.append(pltpu.async_copy(xls[kk].at[sbuf64], bufl[kk], sem0))
                cps.append(pltpu.async_copy(xrs[kk].at[dbuf64], bufr[kk], sem1))
            for cp in cps:
                cp.wait()
            for h in range(HEADS):
                accs = (jnp.zeros((16,), jnp.float32),) * 4
                for kk in (2 * h, 2 * h + 1):
                    def cbody(ci, accs, kk=kk):
                        out = list(accs)
                        for cc in range(8):
                            c = ci * 8 + cc
                            csp = jnp.full((16,), c, jnp.int32)
                            attb = plsc.load_gather(att_v, [csp + (kk * SLAB)])
                            for g in range(4):
                                vl = plsc.load_gather(bufl[kk], [evecs[g], csp])
                                vr = plsc.load_gather(bufr[kk], [evecs[g], csp])
                                t = vl + vr
                                out[g] = out[g] + jnp.maximum(t, 0.2 * t) * attb
                        return tuple(out)
                    accs = lax.fori_loop(0, SLAB // 8, cbody, accs)
                mloc = mlocs[h]
                for g in range(4):
                    lgvs[h][pl.ds(off + 16 * g, 16)] = accs[g]
                    dvec = d_v[pl.ds(off + 16 * g, 16)]
                    val = accs[g]

                    def wbody(pend, dvec=dvec, val=val, mloc=mloc):
                        msk = pend != 0
                        cur = plsc.load_gather(mloc, [dvec])
                        new = jnp.maximum(cur, val)
                        plsc.store_scatter(mloc, [dvec], new, mask=msk)
                        chk = plsc.load_gather(mloc, [dvec])
                        return jnp.where(msk & (chk < new), 1, 0).astype(jnp.int32)

                    lax.while_loop(lambda p: jnp.max(p, axis=0) > 0, wbody, ones16)
            return 0
        lax.fori_loop(0, CH_A // 64, chunk, 0)

        pltpu.sync_copy(lg0_v, lg0_hbm.at[pl.ds(base, CH_A)])
        pltpu.sync_copy(lg1_v, lg1_hbm.at[pl.ds(base, CH_A)])

        for h in range(HEADS):
            plsc.subcore_barrier()
            pltpu.sync_copy(mlocs[h], stage.at[sid])
            plsc.subcore_barrier()
            rbase = sid * NSTRIPE
            pltpu.sync_copy(stage.at[0, pl.ds(rbase, NSTRIPE)], red_acc)

            def tred(t, _):
                pltpu.sync_copy(stage.at[t, pl.ds(rbase, NSTRIPE)], red_tmp)

                def rstep(j, _):
                    red_acc[pl.ds(j * 16, 16)] = jnp.maximum(
                        red_acc[pl.ds(j * 16, 16)], red_tmp[pl.ds(j * 16, 16)])
                    return 0
                lax.fori_loop(0, NSTRIPE // 16, rstep, 0)
                return 0
            lax.fori_loop(1, 16, tred, 0)
            pltpu.sync_copy(red_acc, mpart_hbm.at[cid, h, pl.ds(rbase, NSTRIPE)])

    return k(*xl_slabs, *xr_slabs, s, d, att)


CH_D = EPAD // 16


def _sc_alpha(d, lg0, lg1, m_part4):
    mesh = plsc.VectorSubcoreMesh(core_axis_name="c", subcore_axis_name="s")
    NR = NPAD // 128

    @functools.partial(
        pl.kernel,
        mesh=mesh,
        compiler_params=pltpu.CompilerParams(needs_layout_passes=False),
        out_type=jax.ShapeDtypeStruct((2, EPAD), jnp.float32),
        scratch_types=[
            pltpu.VMEM((CH_D,), jnp.float32),
            pltpu.VMEM((NR, 128), jnp.float32),
            pltpu.VMEM((NR, 128), jnp.float32),
            pltpu.VMEM((64,), jnp.int32),
            pltpu.VMEM((64,), jnp.float32),
            pltpu.VMEM((NR,), jnp.int32),
            pltpu.VMEM_SHARED((NR, 128), jnp.float32),
        ],
    )
    def k(d_hbm, lg0_hbm, lg1_hbm, mpart_hbm, alpha_hbm,
          a_v, m_v, zloc, dbuf, lgbuf, idrows, zfin_sh):
        cid = lax.axis_index("c")
        sid = lax.axis_index("s")
        base = sid * CH_D
        lghs = (lg0_hbm, lg1_hbm)
        iota = jnp.arange(16, dtype=jnp.int32)

        def idf(i, _):
            idrows[pl.ds(i * 16, 16)] = iota + i * 16
            return 0
        lax.fori_loop(0, NR // 16, idf, 0)

        for h in range(HEADS):
            @pl.when(cid == h)
            def _(h=h):
                pltpu.sync_copy(mpart_hbm.at[0, h], m_v)
                pltpu.sync_copy(mpart_hbm.at[1, h], zloc)

                def mmax(j, _):
                    def mc(i, _):
                        m_v[j, pl.ds(i * 16, 16)] = jnp.maximum(
                            m_v[j, pl.ds(i * 16, 16)], zloc[j, pl.ds(i * 16, 16)])
                        return 0
                    lax.fori_loop(0, 8, mc, 0)
                    return 0
                lax.fori_loop(0, NR, mmax, 0)

                def zz(j, _):
                    def zc(i, _):
                        zloc[j, pl.ds(i * 16, 16)] = jnp.zeros((16,), jnp.float32)
                        return 0
                    lax.fori_loop(0, 8, zc, 0)
                    return 0
                lax.fori_loop(0, NR, zz, 0)

                @pl.when(sid == 0)
                def _():
                    pltpu.sync_copy(zloc, zfin_sh)
                plsc.subcore_barrier()

                def abody(ci, _):
                    off = ci * 64
                    pltpu.sync_copy(d_hbm.at[pl.ds(base + off, 64)], dbuf)
                    pltpu.sync_copy(lghs[h].at[pl.ds(base + off, 64)], lgbuf)
                    for g in range(4):
                        dvec = dbuf[pl.ds(16 * g, 16)]
                        hi = lax.shift_right_logical(dvec, 7)
                        lo = lax.bitwise_and(dvec, 127)
                        lg = lgbuf[pl.ds(16 * g, 16)]
                        mg = plsc.load_gather(m_v, [hi, lo])
                        a = jnp.exp(lg - mg)
                        a_v[pl.ds(off + 16 * g, 16)] = a
                        plsc.addupdate_scatter(zloc, [hi, lo], a)
                    return 0
                lax.fori_loop(0, CH_D // 64, abody, 0)

                pltpu.sync_copy(zloc, zfin_sh.at[idrows], add=True)
                plsc.subcore_barrier()
                pltpu.sync_copy(zfin_sh, m_v)

                def albody(ci, _):
                    off = ci * 64
                    pltpu.sync_copy(d_hbm.at[pl.ds(base + off, 64)], dbuf)
                    for g in range(4):
                        dvec = dbuf[pl.ds(16 * g, 16)]
                        hi = lax.shift_right_logical(dvec, 7)
                        lo = lax.bitwise_and(dvec, 127)
                        zg = plsc.load_gather(m_v, [hi, lo])
                        o = off + 16 * g
                        a_v[pl.ds(o, 16)] = a_v[pl.ds(o, 16)] / (zg + 1e-16)
                    return 0
                lax.fori_loop(0, CH_D // 64, albody, 0)
                pltpu.sync_copy(a_v, alpha_hbm.at[h, pl.ds(base, CH_D)])

    return k(d, lg0, lg1, m_part4)


SUP = 1536
NSUP = CH_D // SUP
CHK = 64
CPS = SUP // CHK


def _sc_scatter(xl_slabs, s, d, alpha):
    mesh = plsc.VectorSubcoreMesh(core_axis_name="c", subcore_axis_name="s")
    NG = CHK // 16

    @functools.partial(
        pl.kernel,
        mesh=mesh,
        compiler_params=pltpu.CompilerParams(needs_layout_passes=False),
        out_type=[jax.ShapeDtypeStruct((NPAD, SLAB), jnp.float32)
                  for _ in range(NSLAB)],
        scratch_types=[
            [pltpu.VMEM((CHK, SLAB), jnp.float32) for _ in range(2)],
            [pltpu.VMEM((CHK, SLAB), jnp.float32) for _ in range(2)],
            [pltpu.VMEM((CHK,), jnp.int32) for _ in range(2)],
            [pltpu.VMEM((CHK,), jnp.int32) for _ in range(2)],
            pltpu.VMEM((SUP,), jnp.int32),
            pltpu.VMEM((SUP,), jnp.int32),
            pltpu.VMEM((SUP,), jnp.float32),
            pltpu.VMEM_SHARED((NPAD, SLAB), jnp.float32),
            pltpu.SemaphoreType.DMA,
            pltpu.SemaphoreType.DMA,
        ],
    )
    def k(xl0, xl1, xl2, xl3, s_hbm, d_hbm, alpha_hbm,
          o0, o1, o2, o3,
          rowbuf, scbuf, sb64, db64, sbig, dbig, abig, out_sh, gsem, ssem):
        cid = lax.axis_index("c")
        sid = lax.axis_index("s")
        base = sid * CH_D
        rbase = sid * NSTRIPE
        xls = (xl0, xl1, xl2, xl3)
        outs = (o0, o1, o2, o3)
        iota = jnp.arange(16, dtype=jnp.int32)
        evecs = [iota + 16 * g for g in range(NG)]

        for h in range(HEADS):
            @pl.when(cid == h)
            def _(h=h):
                for kk in (2 * h, 2 * h + 1):
                    def zrow(r, _):
                        def zc(j, _):
                            rowbuf[0][r, pl.ds(j * 16, 16)] = jnp.zeros(
                                (16,), jnp.float32)
                            return 0
                        lax.fori_loop(0, SLAB // 16, zc, 0)
                        return 0
                    lax.fori_loop(0, CHK, zrow, 0)

                    def zcp(r, _):
                        pltpu.sync_copy(rowbuf[0],
                                        out_sh.at[pl.ds(rbase + r * CHK, CHK)])
                        return 0
                    lax.fori_loop(0, NSTRIPE // CHK, zcp, 0)
                    plsc.subcore_barrier()

                    def fire(i, p, kk=kk):
                        off = i * CHK
                        for g in range(NG):
                            sb64[p][pl.ds(16 * g, 16)] = sbig[pl.ds(off + 16 * g, 16)]
                        return pltpu.async_copy(xls[kk].at[sb64[p]], rowbuf[p],
                                                gsem)

                    def wait_g(p, kk=kk):
                        pltpu.make_async_copy(xls[kk].at[sb64[p]], rowbuf[p],
                                              gsem).wait()

                    def drain_s(p):
                        pltpu.make_async_copy(scbuf[p], out_sh.at[db64[p]],
                                              ssem).wait()

                    def step(i, p, first=False, last=False):
                        off = i * CHK
                        wait_g(p)
                        if not first:
                            drain_s(p)
                        als = [abig[pl.ds(off + 16 * g, 16)] for g in range(NG)]
                        for g in range(NG):
                            db64[p][pl.ds(16 * g, 16)] = dbig[pl.ds(off + 16 * g, 16)]

                        def cbody(ci, _):
                            for cc in range(8):
                                csp = jnp.full((16,), ci * 8 + cc, jnp.int32)
                                for g in range(NG):
                                    v = plsc.load_gather(rowbuf[p],
                                                         [evecs[g], csp])
                                    plsc.store_scatter(scbuf[p],
                                                       [evecs[g], csp],
                                                       v * als[g])
                            return 0
                        lax.fori_loop(0, SLAB // 8, cbody, 0)
                        if not last:
                            fire(i + 2, p)
                        pltpu.async_copy(scbuf[p], out_sh.at[db64[p]], ssem,
                                         add=True)

                    def super_body(sup, _, kk=kk):
                        soff = sup * SUP
                        pltpu.sync_copy(s_hbm.at[pl.ds(base + soff, SUP)], sbig)
                        pltpu.sync_copy(d_hbm.at[pl.ds(base + soff, SUP)], dbig)
                        pltpu.sync_copy(alpha_hbm.at[h, pl.ds(base + soff, SUP)],
                                        abig)
                        fire(0, 0)
                        fire(1, 1)
                        step(0, 0, first=True)
                        step(1, 1, first=True)

                        def pair(cj, _):
                            step(cj * 2, 0)
                            step(cj * 2 + 1, 1)
                            return 0
                        lax.fori_loop(1, CPS // 2 - 1, pair, 0)
                        step(CPS - 2, 0, last=True)
                        step(CPS - 1, 1, last=True)
                        drain_s(0)
                        drain_s(1)
                        return 0
                    lax.fori_loop(0, NSUP, super_body, 0)
                    plsc.subcore_barrier()
                    pltpu.sync_copy(out_sh.at[pl.ds(rbase, NSTRIPE)],
                                    outs[kk].at[pl.ds(rbase, NSTRIPE)])
                    plsc.subcore_barrier()

    return k(*xl_slabs, s, d, alpha)


def _bias_add(x, b):
    M, Nc = x.shape

    def body(x_ref, b_ref, o_ref):
        o_ref[...] = x_ref[...] + b_ref[...]

    return pl.pallas_call(
        body,
        grid=(pl.cdiv(M, 1024),),
        in_specs=[
            pl.BlockSpec((1024, Nc), lambda i: (i, 0)),
            pl.BlockSpec((1, Nc), lambda i: (0, 0)),
        ],
        out_specs=pl.BlockSpec((1024, Nc), lambda i: (i, 0)),
        out_shape=jax.ShapeDtypeStruct((M, Nc), jnp.float32),
    )(x, b.reshape(1, Nc))


def _pad_rows(a, rows):
    return jnp.pad(a, ((0, rows - a.shape[0]), (0, 0)))


def _slabs(a):
    ap = _pad_rows(a, NPAD)
    return [ap[:, i * SLAB:(i + 1) * SLAB] for i in range(NSLAB)]


def _slabs64(a):
    ap = _pad_rows(a, NPAD)
    hs = SLAB // 2
    return [ap[:, i * hs:(i + 1) * hs] for i in range(2 * NSLAB)]


def _gat_sc(xl, xr, s, d, att):
    xl_s = _slabs(xl)
    xr_s = _slabs(xr)
    attf = att.reshape(D)
    lg0, lg1, m_part = _sc_logits(xl_s, xr_s, s, d, attf)
    alpha = _sc_alpha(d, lg0, lg1, m_part.reshape(2, 2, NPAD // 128, 128))
    outs = _sc_scatter(xl_s, s, d, alpha)
    return jnp.concatenate(outs, axis=1)[:N]


def kernel(x, edge_index, W1l, W1r, a1, b1, Wlin, blin, W2l, W2r, a2, b2):
    src, dst = edge_index[0], edge_index[1]
    loop = jnp.arange(N, dtype=src.dtype)
    npad_e = EPAD - (E + N)
    s = jnp.concatenate([src, loop, jnp.zeros((npad_e,), src.dtype)])
    d = jnp.concatenate([dst, loop, jnp.full((npad_e,), N, dst.dtype)])
    s = s.astype(jnp.int32)
    d = d.astype(jnp.int32)

    zeros512 = jnp.zeros((D,), jnp.float32)
    xl1 = _mm(x, W1l, zeros512, None)
    xr1 = _mm(x, W1r, zeros512, None)
    g1 = _gat_sc(xl1, xr1, s, d, a1)
    h2 = _mm(g1, Wlin, blin, b1)
    xl2 = _mm(h2, W2l, zeros512, None)
    xr2 = _mm(h2, W2r, zeros512, None)
    g2 = _gat_sc(xl2, xr2, s, d, a2)
    return _bias_add(g2, b2)
